# Initial kernel scaffold; baseline (speedup 1.0000x reference)
import jax
import jax.numpy as jnp
from jax.experimental import pallas as pl


def kernel(x, H, edge_weights, W1, b1, W2, b2, Wc, bc):
    raise NotImplementedError("write your pallas kernel here")


# speedup vs baseline: 530.0298x; 530.0298x over previous
import jax
import jax.numpy as jnp
from jax.experimental import pallas as pl
from jax.experimental.pallas import tpu as pltpu

_BLOCK = 1000


def _pass1_kernel(x_ref, H_ref, W1_ref, m1_ref, binv_ref, acc_ref, bdeg_ref):
    i = pl.program_id(0)
    nb = pl.num_programs(0)

    @pl.when(i == 0)
    def _():
        acc_ref[...] = jnp.zeros_like(acc_ref)
        bdeg_ref[...] = jnp.zeros_like(bdeg_ref)

    Hf = H_ref[...].astype(jnp.float32)
    xw = jnp.dot(x_ref[...], W1_ref[...], preferred_element_type=jnp.float32)
    acc_ref[...] += jax.lax.dot_general(
        Hf, xw, (((0,), (0,)), ((), ())), preferred_element_type=jnp.float32)
    ones = jnp.ones((Hf.shape[0], 1), jnp.float32)
    bdeg_ref[...] += jax.lax.dot_general(
        Hf, ones, (((0,), (0,)), ((), ())), preferred_element_type=jnp.float32)

    @pl.when(i == nb - 1)
    def _():
        bdeg = bdeg_ref[...]
        binv = jnp.where(bdeg > 0, 1.0 / bdeg, 0.0)
        binv_ref[...] = binv
        m1_ref[...] = binv * acc_ref[...]


def _pass2_kernel(H_ref, m1_ref, W2_ref, b1_ref, binv_ref, m2_ref, acc_ref):
    i = pl.program_id(0)
    nb = pl.num_programs(0)

    @pl.when(i == 0)
    def _():
        acc_ref[...] = jnp.zeros_like(acc_ref)

    Hf = H_ref[...].astype(jnp.float32)
    deg = jnp.sum(Hf, axis=1, keepdims=True)
    dinv = jnp.where(deg > 0, 1.0 / deg, 0.0)
    h = jax.nn.relu(
        dinv * jnp.dot(Hf, m1_ref[...], preferred_element_type=jnp.float32)
        + b1_ref[...])
    hw = jnp.dot(h, W2_ref[...], preferred_element_type=jnp.float32)
    acc_ref[...] += jax.lax.dot_general(
        Hf, hw, (((0,), (0,)), ((), ())), preferred_element_type=jnp.float32)

    @pl.when(i == nb - 1)
    def _():
        m2_ref[...] = binv_ref[...] * acc_ref[...]


def _pass3_kernel(H_ref, m2_ref, b2_ref, Wc_ref, bc_ref, out_ref):
    Hf = H_ref[...].astype(jnp.float32)
    deg = jnp.sum(Hf, axis=1, keepdims=True)
    dinv = jnp.where(deg > 0, 1.0 / deg, 0.0)
    h = jax.nn.relu(
        dinv * jnp.dot(Hf, m2_ref[...], preferred_element_type=jnp.float32)
        + b2_ref[...])
    out_ref[...] = (
        jnp.dot(h, Wc_ref[...], preferred_element_type=jnp.float32)
        + bc_ref[...])


def kernel(x, H, edge_weights, W1, b1, W2, b2, Wc, bc):
    del edge_weights
    n, d_in = x.shape
    e_h = H.shape[1]
    d_hid = W1.shape[1]
    nb = n // _BLOCK

    full = lambda *shape: pl.BlockSpec(shape, lambda i: (0,) * len(shape))
    row_block = lambda cols: pl.BlockSpec((_BLOCK, cols), lambda i: (i, 0))

    m1, binv = pl.pallas_call(
        _pass1_kernel,
        grid=(nb,),
        in_specs=[row_block(d_in), row_block(e_h), full(d_in, d_hid)],
        out_specs=[full(e_h, d_hid), full(e_h, 1)],
        out_shape=[
            jax.ShapeDtypeStruct((e_h, d_hid), jnp.float32),
            jax.ShapeDtypeStruct((e_h, 1), jnp.float32),
        ],
        scratch_shapes=[
            pltpu.VMEM((e_h, d_hid), jnp.float32),
            pltpu.VMEM((e_h, 1), jnp.float32),
        ],
        compiler_params=pltpu.CompilerParams(
            dimension_semantics=("arbitrary",)),
    )(x, H, W1)

    m2 = pl.pallas_call(
        _pass2_kernel,
        grid=(nb,),
        in_specs=[row_block(e_h), full(e_h, d_hid), full(d_hid, d_hid),
                  full(1, d_hid), full(e_h, 1)],
        out_specs=full(e_h, d_hid),
        out_shape=jax.ShapeDtypeStruct((e_h, d_hid), jnp.float32),
        scratch_shapes=[pltpu.VMEM((e_h, d_hid), jnp.float32)],
        compiler_params=pltpu.CompilerParams(
            dimension_semantics=("arbitrary",)),
    )(H, m1, W2, b1.reshape(1, d_hid), binv)

    out = pl.pallas_call(
        _pass3_kernel,
        grid=(nb,),
        in_specs=[row_block(e_h), full(e_h, d_hid), full(1, d_hid),
                  full(d_hid, 1), full(1, 1)],
        out_specs=row_block(1),
        out_shape=jax.ShapeDtypeStruct((n, 1), jnp.float32),
        compiler_params=pltpu.CompilerParams(
            dimension_semantics=("arbitrary",)),
    )(H, m2, b2.reshape(1, d_hid), Wc, bc.reshape(1, 1))

    return out


# speedup vs baseline: 674.6295x; 1.2728x over previous
import jax
import jax.numpy as jnp
from jax.experimental import pallas as pl
from jax.experimental.pallas import tpu as pltpu

_BLOCK = 2000


def _hgnn_kernel(x_ref, H_ref, W1_ref, W2_ref, b1_ref, b2_ref, Wc_ref,
                 bc_ref, out_ref, acc_ref, bdeg_ref, binv_ref, m_ref):
    p = pl.program_id(0)
    i = pl.program_id(1)
    nb = pl.num_programs(1)
    last = i == nb - 1

    @pl.when(i == 0)
    def _():
        acc_ref[...] = jnp.zeros_like(acc_ref)

    Hf = H_ref[...].astype(jnp.float32)

    @pl.when(p == 0)
    def _():
        @pl.when(i == 0)
        def _():
            bdeg_ref[...] = jnp.zeros_like(bdeg_ref)

        xw = jnp.dot(x_ref[...], W1_ref[...],
                     preferred_element_type=jnp.float32)
        acc_ref[...] += jax.lax.dot_general(
            Hf, xw, (((0,), (0,)), ((), ())),
            preferred_element_type=jnp.float32)
        ones = jnp.ones((Hf.shape[0], 1), jnp.float32)
        bdeg_ref[...] += jax.lax.dot_general(
            Hf, ones, (((0,), (0,)), ((), ())),
            preferred_element_type=jnp.float32)

        @pl.when(last)
        def _():
            bdeg = bdeg_ref[...]
            binv = jnp.where(bdeg > 0, 1.0 / bdeg, 0.0)
            binv_ref[...] = binv
            m_ref[...] = binv * acc_ref[...]

    @pl.when(p > 0)
    def _():
        deg = jnp.sum(Hf, axis=1, keepdims=True)
        dinv = jnp.where(deg > 0, 1.0 / deg, 0.0)
        hm = jnp.dot(Hf, m_ref[...], preferred_element_type=jnp.float32)

        @pl.when(p == 1)
        def _():
            h = jax.nn.relu(dinv * hm + b1_ref[...])
            hw = jnp.dot(h, W2_ref[...], preferred_element_type=jnp.float32)
            acc_ref[...] += jax.lax.dot_general(
                Hf, hw, (((0,), (0,)), ((), ())),
                preferred_element_type=jnp.float32)

            @pl.when(last)
            def _():
                m_ref[...] = binv_ref[...] * acc_ref[...]

        @pl.when(p == 2)
        def _():
            h = jax.nn.relu(dinv * hm + b2_ref[...])
            out_ref[...] = (
                jnp.dot(h, Wc_ref[...], preferred_element_type=jnp.float32)
                + bc_ref[...])


def kernel(x, H, edge_weights, W1, b1, W2, b2, Wc, bc):
    del edge_weights
    n, d_in = x.shape
    e_h = H.shape[1]
    d_hid = W1.shape[1]
    nb = n // _BLOCK

    full = lambda *shape: pl.BlockSpec(shape, lambda p, i: (0,) * len(shape))

    out = pl.pallas_call(
        _hgnn_kernel,
        grid=(3, nb),
        in_specs=[
            pl.BlockSpec((_BLOCK, d_in), lambda p, i: (i * (p == 0), 0)),
            pl.BlockSpec((_BLOCK, e_h), lambda p, i: (i, 0)),
            full(d_in, d_hid),
            full(d_hid, d_hid),
            full(1, d_hid),
            full(1, d_hid),
            full(d_hid, 1),
            full(1, 1),
        ],
        out_specs=pl.BlockSpec((_BLOCK, 1), lambda p, i: (i, 0)),
        out_shape=jax.ShapeDtypeStruct((n, 1), jnp.float32),
        scratch_shapes=[
            pltpu.VMEM((e_h, d_hid), jnp.float32),
            pltpu.VMEM((e_h, 1), jnp.float32),
            pltpu.VMEM((e_h, 1), jnp.float32),
            pltpu.VMEM((e_h, d_hid), jnp.float32),
        ],
        compiler_params=pltpu.CompilerParams(
            dimension_semantics=("arbitrary", "arbitrary")),
    )(x, H, W1, W2, b1.reshape(1, d_hid), b2.reshape(1, d_hid), Wc,
      bc.reshape(1, 1))

    return out


# speedup vs baseline: 961.6849x; 1.4255x over previous
import jax
import jax.numpy as jnp
from jax.experimental import pallas as pl
from jax.experimental.pallas import tpu as pltpu


def _hgnn_kernel(x_ref, H_ref, W1_ref, W2_ref, b1_ref, b2_ref, Wc_ref,
                 bc_ref, out_ref):
    f32 = jnp.float32
    Hf = H_ref[...].astype(f32)
    ones = jnp.ones((Hf.shape[0], 1), f32)
    bdeg = jax.lax.dot_general(
        Hf, ones, (((0,), (0,)), ((), ())), preferred_element_type=f32)
    binv = jnp.where(bdeg > 0, 1.0 / bdeg, 0.0)
    deg = jnp.sum(Hf, axis=1, keepdims=True)
    dinv = jnp.where(deg > 0, 1.0 / deg, 0.0)

    xw = jnp.dot(x_ref[...], W1_ref[...], preferred_element_type=f32)
    m = binv * jax.lax.dot_general(
        Hf, xw, (((0,), (0,)), ((), ())), preferred_element_type=f32)
    h = jax.nn.relu(
        dinv * jnp.dot(Hf, m, preferred_element_type=f32) + b1_ref[...])

    hw = jnp.dot(h, W2_ref[...], preferred_element_type=f32)
    m2 = binv * jax.lax.dot_general(
        Hf, hw, (((0,), (0,)), ((), ())), preferred_element_type=f32)
    h2 = jax.nn.relu(
        dinv * jnp.dot(Hf, m2, preferred_element_type=f32) + b2_ref[...])

    out_ref[...] = (
        jnp.dot(h2, Wc_ref[...], preferred_element_type=f32) + bc_ref[...])


def kernel(x, H, edge_weights, W1, b1, W2, b2, Wc, bc):
    del edge_weights
    n, d_in = x.shape
    d_hid = W1.shape[1]

    out = pl.pallas_call(
        _hgnn_kernel,
        out_shape=jax.ShapeDtypeStruct((n, 1), jnp.float32),
    )(x, H, W1, W2, b1.reshape(1, d_hid), b2.reshape(1, d_hid), Wc,
      bc.reshape(1, 1))

    return out
